# baseline (speedup 1.0000x reference)
import functools

import jax
import jax.numpy as jnp
from jax import lax
from jax.experimental import pallas as pl
from jax.experimental.pallas import tpu as pltpu
from jax.experimental.pallas import tpu_sc as plsc

B = 16384
EMB = 16
HID = 64
NB = 1000
NM = 1000
NSLOT = 7
XW = 128

NCK = 4
CB = B // NCK

NC = 2
NS = 16
NW = NC * NS
BPW = CB // NW
CHUNK = 128
NGRP = BPW // CHUNK


def _sc_gather_body(tab_hbm, idx_hbm, out_hbm, idx_v, out_v, sem):
    cid = lax.axis_index("c")
    sid = lax.axis_index("s")
    wid = sid * NC + cid

    pltpu.sync_copy(idx_hbm.at[:, pl.ds(wid * BPW, BPW)], idx_v)

    descs = []
    for g in range(NGRP):
        descs.append(
            pltpu.async_copy(
                tab_hbm.at[idx_v.at[0, pl.ds(g * CHUNK, CHUNK)], :],
                out_v.at[pl.ds(g * CHUNK, CHUNK), :],
                sem,
            )
        )
    for d in descs:
        d.wait()

    descs = []
    for g in range(NGRP):
        for s in range(1, NSLOT):
            descs.append(
                pltpu.async_copy(
                    tab_hbm.at[idx_v.at[s, pl.ds(g * CHUNK, CHUNK)], :],
                    out_v.at[pl.ds(g * CHUNK, CHUNK), :],
                    sem,
                    add=True,
                )
            )
    for d in descs:
        d.wait()

    pltpu.sync_copy(out_v, out_hbm.at[pl.ds(wid * BPW, BPW), :])


_sc_gather = functools.partial(
    pl.kernel,
    out_type=jax.ShapeDtypeStruct((CB, XW), jnp.float32),
    mesh=plsc.VectorSubcoreMesh(core_axis_name="c", subcore_axis_name="s"),
    scratch_types=[
        pltpu.VMEM((NSLOT + 1, BPW), jnp.int32),
        pltpu.VMEM((BPW, XW), jnp.float32),
        pltpu.SemaphoreType.DMA,
    ],
)(_sc_gather_body)


BLK = 1024
BPC = CB // BLK


def _mlp_body(x_ref, w1_ref, b1_ref, w2_ref, b2_ref, o_ref):
    h = jnp.dot(x_ref[...], w1_ref[...], preferred_element_type=jnp.float32)
    h = jnp.maximum(h + b1_ref[...], 0.0)
    o_ref[...] = (
        jnp.dot(
            h.astype(jnp.bfloat16),
            w2_ref[...],
            preferred_element_type=jnp.float32,
        )
        + b2_ref[...]
    )


def _mlp_body_carry(x_ref, w1_ref, b1_ref, w2_ref, b2_ref, carry_ref, o_ref):
    del carry_ref
    _mlp_body(x_ref, w1_ref, b1_ref, w2_ref, b2_ref, o_ref)


def _tc_mlp_chunk(k, x, w1p, b1r, w2t, b2r, carry):
    in_specs = [
        pl.BlockSpec((BLK, XW), lambda i: (i, 0)),
        pl.BlockSpec((XW, HID), lambda i: (0, 0)),
        pl.BlockSpec((1, HID), lambda i: (0, 0)),
        pl.BlockSpec((HID, NB), lambda i: (0, 0)),
        pl.BlockSpec((1, NB), lambda i: (0, 0)),
    ]
    args = (x, w1p, b1r, w2t, b2r)
    if carry is None:
        body = _mlp_body
        aliases = {}
    else:
        body = _mlp_body_carry
        in_specs = in_specs + [pl.BlockSpec(memory_space=pl.ANY)]
        args = args + (carry,)
        aliases = {5: 0}
    return pl.pallas_call(
        body,
        out_shape=jax.ShapeDtypeStruct((B, NB), jnp.float32),
        grid=(BPC,),
        in_specs=in_specs,
        out_specs=pl.BlockSpec((BLK, NB), lambda i, _k=k: (i + _k * BPC, 0)),
        input_output_aliases=aliases,
        compiler_params=pltpu.CompilerParams(
            dimension_semantics=("parallel",),
            vmem_limit_bytes=100 * 1024 * 1024,
        ),
    )(*args)


def kernel(friends, enemies, map_idx, brawler_emb, map_emb, W1, b1, W2, b2):
    tab7 = jnp.concatenate(
        [
            jnp.pad(brawler_emb, ((0, 0), (16 * s, XW - EMB - 16 * s)))
            for s in range(6)
        ]
        + [jnp.pad(map_emb, ((0, 0), (16 * 6, XW - EMB - 16 * 6)))],
        axis=0,
    )

    off_f = (jnp.arange(3, dtype=jnp.int32) * NB)[:, None]
    off_e = ((3 + jnp.arange(3, dtype=jnp.int32)) * NB)[:, None]
    idx_sm = jnp.concatenate(
        [
            friends.astype(jnp.int32).T + off_f,
            enemies.astype(jnp.int32).T + off_e,
            map_idx.astype(jnp.int32).T + 6 * NB,
            jnp.zeros((1, B), jnp.int32),
        ],
        axis=0,
    )

    w1p = jnp.concatenate(
        [W1.T, jnp.zeros((XW - W1.shape[1], HID), jnp.float32)], axis=0
    )
    b1r = b1.reshape(1, HID)
    w2t = W2.T.astype(jnp.bfloat16)
    b2r = b2.reshape(1, NB)

    xs = [
        _sc_gather(tab7, lax.slice(idx_sm, (0, k * CB), (NSLOT + 1, (k + 1) * CB)))
        for k in range(NCK)
    ]
    out = None
    for k in range(NCK):
        out = _tc_mlp_chunk(k, xs[k], w1p, b1r, w2t, b2r, out)
    return out


# speedup vs baseline: 1.0334x; 1.0334x over previous
import functools

import jax
import jax.numpy as jnp
from jax import lax
from jax.experimental import pallas as pl
from jax.experimental.pallas import tpu as pltpu
from jax.experimental.pallas import tpu_sc as plsc

B = 16384
EMB = 16
HID = 64
NB = 1000
NM = 1000
NSLOT = 7
XW = 128

NCK = 2
CB = B // NCK

NC = 2
NS = 16
NW = NC * NS
BPW = CB // NW
CHUNK = 128
NGRP = BPW // CHUNK


def _sc_gather_body(tab_hbm, idx_hbm, out_hbm, idx_v, out_v, sem):
    cid = lax.axis_index("c")
    sid = lax.axis_index("s")
    wid = sid * NC + cid

    pltpu.sync_copy(idx_hbm.at[:, pl.ds(wid * BPW, BPW)], idx_v)

    descs = []
    for g in range(NGRP):
        descs.append(
            pltpu.async_copy(
                tab_hbm.at[idx_v.at[0, pl.ds(g * CHUNK, CHUNK)], :],
                out_v.at[pl.ds(g * CHUNK, CHUNK), :],
                sem,
            )
        )
    for d in descs:
        d.wait()

    descs = []
    for g in range(NGRP):
        for s in range(1, NSLOT):
            descs.append(
                pltpu.async_copy(
                    tab_hbm.at[idx_v.at[s, pl.ds(g * CHUNK, CHUNK)], :],
                    out_v.at[pl.ds(g * CHUNK, CHUNK), :],
                    sem,
                    add=True,
                )
            )
    for d in descs:
        d.wait()

    pltpu.sync_copy(out_v, out_hbm.at[pl.ds(wid * BPW, BPW), :])


_sc_gather = functools.partial(
    pl.kernel,
    out_type=jax.ShapeDtypeStruct((CB, XW), jnp.float32),
    mesh=plsc.VectorSubcoreMesh(core_axis_name="c", subcore_axis_name="s"),
    scratch_types=[
        pltpu.VMEM((NSLOT + 1, BPW), jnp.int32),
        pltpu.VMEM((BPW, XW), jnp.float32),
        pltpu.SemaphoreType.DMA,
    ],
)(_sc_gather_body)


BLK = 1024
BPC = CB // BLK


def _mlp_body(x_ref, w1_ref, b1_ref, w2_ref, b2_ref, o_ref):
    h = jnp.dot(x_ref[...], w1_ref[...], preferred_element_type=jnp.float32)
    h = jnp.maximum(h + b1_ref[...], 0.0)
    o_ref[...] = (
        jnp.dot(
            h.astype(jnp.bfloat16),
            w2_ref[...],
            preferred_element_type=jnp.float32,
        )
        + b2_ref[...]
    )


def _mlp_body_carry(x_ref, w1_ref, b1_ref, w2_ref, b2_ref, carry_ref, o_ref):
    del carry_ref
    _mlp_body(x_ref, w1_ref, b1_ref, w2_ref, b2_ref, o_ref)


def _tc_mlp_chunk(k, x, w1p, b1r, w2t, b2r, carry):
    in_specs = [
        pl.BlockSpec((BLK, XW), lambda i: (i, 0)),
        pl.BlockSpec((XW, HID), lambda i: (0, 0)),
        pl.BlockSpec((1, HID), lambda i: (0, 0)),
        pl.BlockSpec((HID, NB), lambda i: (0, 0)),
        pl.BlockSpec((1, NB), lambda i: (0, 0)),
    ]
    args = (x, w1p, b1r, w2t, b2r)
    if carry is None:
        body = _mlp_body
        aliases = {}
    else:
        body = _mlp_body_carry
        in_specs = in_specs + [pl.BlockSpec(memory_space=pl.ANY)]
        args = args + (carry,)
        aliases = {5: 0}
    return pl.pallas_call(
        body,
        out_shape=jax.ShapeDtypeStruct((B, NB), jnp.float32),
        grid=(BPC,),
        in_specs=in_specs,
        out_specs=pl.BlockSpec((BLK, NB), lambda i, _k=k: (i + _k * BPC, 0)),
        input_output_aliases=aliases,
        compiler_params=pltpu.CompilerParams(
            dimension_semantics=("parallel",),
            vmem_limit_bytes=100 * 1024 * 1024,
        ),
    )(*args)


def kernel(friends, enemies, map_idx, brawler_emb, map_emb, W1, b1, W2, b2):
    tab7 = jnp.concatenate(
        [
            jnp.pad(brawler_emb, ((0, 0), (16 * s, XW - EMB - 16 * s)))
            for s in range(6)
        ]
        + [jnp.pad(map_emb, ((0, 0), (16 * 6, XW - EMB - 16 * 6)))],
        axis=0,
    )

    off_f = (jnp.arange(3, dtype=jnp.int32) * NB)[:, None]
    off_e = ((3 + jnp.arange(3, dtype=jnp.int32)) * NB)[:, None]
    idx_sm = jnp.concatenate(
        [
            friends.astype(jnp.int32).T + off_f,
            enemies.astype(jnp.int32).T + off_e,
            map_idx.astype(jnp.int32).T + 6 * NB,
            jnp.zeros((1, B), jnp.int32),
        ],
        axis=0,
    )

    w1p = jnp.concatenate(
        [W1.T, jnp.zeros((XW - W1.shape[1], HID), jnp.float32)], axis=0
    )
    b1r = b1.reshape(1, HID)
    w2t = W2.T.astype(jnp.bfloat16)
    b2r = b2.reshape(1, NB)

    xs = [
        _sc_gather(tab7, lax.slice(idx_sm, (0, k * CB), (NSLOT + 1, (k + 1) * CB)))
        for k in range(NCK)
    ]
    out = None
    for k in range(NCK):
        out = _tc_mlp_chunk(k, xs[k], w1p, b1r, w2t, b2r, out)
    return out


# speedup vs baseline: 1.0840x; 1.0490x over previous
import functools

import jax
import jax.numpy as jnp
from jax import lax
from jax.experimental import pallas as pl
from jax.experimental.pallas import tpu as pltpu
from jax.experimental.pallas import tpu_sc as plsc

B = 16384
EMB = 16
HID = 64
NB = 1000
NM = 1000
NSLOT = 7
XW = 128

NCK = 1
CB = B // NCK

NC = 2
NS = 16
NW = NC * NS
BPW = CB // NW
CHUNK = 128
NGRP = BPW // CHUNK


def _sc_gather_body(tab_hbm, idx_hbm, out_hbm, idx_v, out_v, sem):
    cid = lax.axis_index("c")
    sid = lax.axis_index("s")
    wid = sid * NC + cid

    pltpu.sync_copy(idx_hbm.at[:, pl.ds(wid * BPW, BPW)], idx_v)

    descs = []
    for g in range(NGRP):
        descs.append(
            pltpu.async_copy(
                tab_hbm.at[idx_v.at[0, pl.ds(g * CHUNK, CHUNK)], :],
                out_v.at[pl.ds(g * CHUNK, CHUNK), :],
                sem,
            )
        )
    for d in descs:
        d.wait()

    descs = []
    for g in range(NGRP):
        for s in range(1, NSLOT):
            descs.append(
                pltpu.async_copy(
                    tab_hbm.at[idx_v.at[s, pl.ds(g * CHUNK, CHUNK)], :],
                    out_v.at[pl.ds(g * CHUNK, CHUNK), :],
                    sem,
                    add=True,
                )
            )
    for d in descs:
        d.wait()

    pltpu.sync_copy(out_v, out_hbm.at[pl.ds(wid * BPW, BPW), :])


_sc_gather = functools.partial(
    pl.kernel,
    out_type=jax.ShapeDtypeStruct((CB, XW), jnp.float32),
    mesh=plsc.VectorSubcoreMesh(core_axis_name="c", subcore_axis_name="s"),
    scratch_types=[
        pltpu.VMEM((NSLOT + 1, BPW), jnp.int32),
        pltpu.VMEM((BPW, XW), jnp.float32),
        pltpu.SemaphoreType.DMA,
    ],
)(_sc_gather_body)


BLK = 4096
BPC = CB // BLK


def _mlp_body(x_ref, w1_ref, b1_ref, w2_ref, b2_ref, o_ref):
    h = jnp.dot(x_ref[...], w1_ref[...], preferred_element_type=jnp.float32)
    h = jnp.maximum(h + b1_ref[...], 0.0)
    o_ref[...] = (
        jnp.dot(
            h.astype(jnp.bfloat16),
            w2_ref[...],
            preferred_element_type=jnp.float32,
        )
        + b2_ref[...]
    )


def _mlp_body_carry(x_ref, w1_ref, b1_ref, w2_ref, b2_ref, carry_ref, o_ref):
    del carry_ref
    _mlp_body(x_ref, w1_ref, b1_ref, w2_ref, b2_ref, o_ref)


def _tc_mlp_chunk(k, x, w1p, b1r, w2t, b2r, carry):
    in_specs = [
        pl.BlockSpec((BLK, XW), lambda i: (i, 0)),
        pl.BlockSpec((XW, HID), lambda i: (0, 0)),
        pl.BlockSpec((1, HID), lambda i: (0, 0)),
        pl.BlockSpec((HID, NB), lambda i: (0, 0)),
        pl.BlockSpec((1, NB), lambda i: (0, 0)),
    ]
    args = (x, w1p, b1r, w2t, b2r)
    if carry is None:
        body = _mlp_body
        aliases = {}
    else:
        body = _mlp_body_carry
        in_specs = in_specs + [pl.BlockSpec(memory_space=pl.ANY)]
        args = args + (carry,)
        aliases = {5: 0}
    return pl.pallas_call(
        body,
        out_shape=jax.ShapeDtypeStruct((B, NB), jnp.float32),
        grid=(BPC,),
        in_specs=in_specs,
        out_specs=pl.BlockSpec((BLK, NB), lambda i, _k=k: (i + _k * BPC, 0)),
        input_output_aliases=aliases,
        compiler_params=pltpu.CompilerParams(
            dimension_semantics=("parallel",),
            vmem_limit_bytes=100 * 1024 * 1024,
        ),
    )(*args)


def kernel(friends, enemies, map_idx, brawler_emb, map_emb, W1, b1, W2, b2):
    tab7 = jnp.concatenate(
        [
            jnp.pad(brawler_emb, ((0, 0), (16 * s, XW - EMB - 16 * s)))
            for s in range(6)
        ]
        + [jnp.pad(map_emb, ((0, 0), (16 * 6, XW - EMB - 16 * 6)))],
        axis=0,
    )

    off_f = (jnp.arange(3, dtype=jnp.int32) * NB)[:, None]
    off_e = ((3 + jnp.arange(3, dtype=jnp.int32)) * NB)[:, None]
    idx_sm = jnp.concatenate(
        [
            friends.astype(jnp.int32).T + off_f,
            enemies.astype(jnp.int32).T + off_e,
            map_idx.astype(jnp.int32).T + 6 * NB,
            jnp.zeros((1, B), jnp.int32),
        ],
        axis=0,
    )

    w1p = jnp.concatenate(
        [W1.T, jnp.zeros((XW - W1.shape[1], HID), jnp.float32)], axis=0
    )
    b1r = b1.reshape(1, HID)
    w2t = W2.T.astype(jnp.bfloat16)
    b2r = b2.reshape(1, NB)

    xs = [
        _sc_gather(tab7, lax.slice(idx_sm, (0, k * CB), (NSLOT + 1, (k + 1) * CB)))
        for k in range(NCK)
    ]
    out = None
    for k in range(NCK):
        out = _tc_mlp_chunk(k, xs[k], w1p, b1r, w2t, b2r, out)
    return out


# speedup vs baseline: 1.0895x; 1.0051x over previous
import functools

import jax
import jax.numpy as jnp
from jax import lax
from jax.experimental import pallas as pl
from jax.experimental.pallas import tpu as pltpu
from jax.experimental.pallas import tpu_sc as plsc

B = 16384
EMB = 16
HID = 64
NB = 1000
NM = 1000
NSLOT = 7
XW = 128

NCK = 1
CB = B // NCK

NC = 2
NS = 16
NW = NC * NS
BPW = CB // NW
CHUNK = 128
NGRP = BPW // CHUNK


def _sc_gather_body(tab_hbm, idx_hbm, out_hbm, idx_v, out_v, sem):
    cid = lax.axis_index("c")
    sid = lax.axis_index("s")
    wid = sid * NC + cid

    pltpu.sync_copy(idx_hbm.at[:, pl.ds(wid * BPW, BPW)], idx_v)

    descs = []
    for g in range(NGRP):
        descs.append(
            pltpu.async_copy(
                tab_hbm.at[idx_v.at[0, pl.ds(g * CHUNK, CHUNK)], :],
                out_v.at[pl.ds(g * CHUNK, CHUNK), :],
                sem,
            )
        )
    for d in descs:
        d.wait()

    descs = []
    for g in range(NGRP):
        for s in range(1, NSLOT):
            descs.append(
                pltpu.async_copy(
                    tab_hbm.at[idx_v.at[s, pl.ds(g * CHUNK, CHUNK)], :],
                    out_v.at[pl.ds(g * CHUNK, CHUNK), :],
                    sem,
                    add=True,
                )
            )
    for d in descs:
        d.wait()

    pltpu.sync_copy(out_v, out_hbm.at[pl.ds(wid * BPW, BPW), :])


_sc_gather = functools.partial(
    pl.kernel,
    out_type=jax.ShapeDtypeStruct((CB, XW), jnp.float32),
    mesh=plsc.VectorSubcoreMesh(core_axis_name="c", subcore_axis_name="s"),
    scratch_types=[
        pltpu.VMEM((NSLOT + 1, BPW), jnp.int32),
        pltpu.VMEM((BPW, XW), jnp.float32),
        pltpu.SemaphoreType.DMA,
    ],
)(_sc_gather_body)


BLK = 2048
BPC = CB // BLK


def _mlp_body(x_ref, w1_ref, b1_ref, w2_ref, b2_ref, o_ref):
    h = jnp.dot(x_ref[...], w1_ref[...], preferred_element_type=jnp.float32)
    h = jnp.maximum(h + b1_ref[...], 0.0)
    o_ref[...] = (
        jnp.dot(
            h.astype(jnp.bfloat16),
            w2_ref[...],
            preferred_element_type=jnp.float32,
        )
        + b2_ref[...]
    )


def _mlp_body_carry(x_ref, w1_ref, b1_ref, w2_ref, b2_ref, carry_ref, o_ref):
    del carry_ref
    _mlp_body(x_ref, w1_ref, b1_ref, w2_ref, b2_ref, o_ref)


def _tc_mlp_chunk(k, x, w1p, b1r, w2t, b2r, carry):
    in_specs = [
        pl.BlockSpec((BLK, XW), lambda i: (i, 0)),
        pl.BlockSpec((XW, HID), lambda i: (0, 0)),
        pl.BlockSpec((1, HID), lambda i: (0, 0)),
        pl.BlockSpec((HID, NB), lambda i: (0, 0)),
        pl.BlockSpec((1, NB), lambda i: (0, 0)),
    ]
    args = (x, w1p, b1r, w2t, b2r)
    if carry is None:
        body = _mlp_body
        aliases = {}
    else:
        body = _mlp_body_carry
        in_specs = in_specs + [pl.BlockSpec(memory_space=pl.ANY)]
        args = args + (carry,)
        aliases = {5: 0}
    return pl.pallas_call(
        body,
        out_shape=jax.ShapeDtypeStruct((B, NB), jnp.float32),
        grid=(BPC,),
        in_specs=in_specs,
        out_specs=pl.BlockSpec((BLK, NB), lambda i, _k=k: (i + _k * BPC, 0)),
        input_output_aliases=aliases,
        compiler_params=pltpu.CompilerParams(
            dimension_semantics=("parallel",),
            vmem_limit_bytes=100 * 1024 * 1024,
        ),
    )(*args)


def kernel(friends, enemies, map_idx, brawler_emb, map_emb, W1, b1, W2, b2):
    tab7 = jnp.concatenate(
        [
            jnp.pad(brawler_emb, ((0, 0), (16 * s, XW - EMB - 16 * s)))
            for s in range(6)
        ]
        + [jnp.pad(map_emb, ((0, 0), (16 * 6, XW - EMB - 16 * 6)))],
        axis=0,
    )

    off_f = (jnp.arange(3, dtype=jnp.int32) * NB)[:, None]
    off_e = ((3 + jnp.arange(3, dtype=jnp.int32)) * NB)[:, None]
    idx_sm = jnp.concatenate(
        [
            friends.astype(jnp.int32).T + off_f,
            enemies.astype(jnp.int32).T + off_e,
            map_idx.astype(jnp.int32).T + 6 * NB,
            jnp.zeros((1, B), jnp.int32),
        ],
        axis=0,
    )

    w1p = jnp.concatenate(
        [W1.T, jnp.zeros((XW - W1.shape[1], HID), jnp.float32)], axis=0
    )
    b1r = b1.reshape(1, HID)
    w2t = W2.T.astype(jnp.bfloat16)
    b2r = b2.reshape(1, NB)

    xs = [
        _sc_gather(tab7, lax.slice(idx_sm, (0, k * CB), (NSLOT + 1, (k + 1) * CB)))
        for k in range(NCK)
    ]
    out = None
    for k in range(NCK):
        out = _tc_mlp_chunk(k, xs[k], w1p, b1r, w2t, b2r, out)
    return out


# speedup vs baseline: 1.0903x; 1.0007x over previous
import functools

import jax
import jax.numpy as jnp
from jax import lax
from jax.experimental import pallas as pl
from jax.experimental.pallas import tpu as pltpu
from jax.experimental.pallas import tpu_sc as plsc

B = 16384
EMB = 16
HID = 64
NB = 1000
NM = 1000
NSLOT = 7
XW = 128

NC = 2
NS = 16
NW = NC * NS
BPW = B // NW
CHUNK = 128
NGRP = BPW // CHUNK


def _sc_gather_body(tab_hbm, idx_hbm, out_hbm, idx_v, out_v, sem):
    cid = lax.axis_index("c")
    sid = lax.axis_index("s")
    wid = sid * NC + cid

    pltpu.sync_copy(idx_hbm.at[:, pl.ds(wid * BPW, BPW)], idx_v)

    descs = []
    for g in range(NGRP):
        descs.append(
            pltpu.async_copy(
                tab_hbm.at[idx_v.at[0, pl.ds(g * CHUNK, CHUNK)], :],
                out_v.at[pl.ds(g * CHUNK, CHUNK), :],
                sem,
            )
        )
    for d in descs:
        d.wait()

    descs = []
    for g in range(NGRP):
        for s in range(1, NSLOT):
            descs.append(
                pltpu.async_copy(
                    tab_hbm.at[idx_v.at[s, pl.ds(g * CHUNK, CHUNK)], :],
                    out_v.at[pl.ds(g * CHUNK, CHUNK), :],
                    sem,
                    add=True,
                )
            )
    for d in descs:
        d.wait()

    pltpu.sync_copy(out_v, out_hbm.at[pl.ds(wid * BPW, BPW), :])


_sc_gather = functools.partial(
    pl.kernel,
    out_type=jax.ShapeDtypeStruct((B, XW), jnp.float32),
    mesh=plsc.VectorSubcoreMesh(core_axis_name="c", subcore_axis_name="s"),
    scratch_types=[
        pltpu.VMEM((NSLOT + 1, BPW), jnp.int32),
        pltpu.VMEM((BPW, XW), jnp.float32),
        pltpu.SemaphoreType.DMA,
    ],
)(_sc_gather_body)


BLK = 2048


def _mlp_body(x_ref, w1_ref, b1_ref, w2_ref, b2_ref, o_ref):
    h = jnp.dot(x_ref[...], w1_ref[...], preferred_element_type=jnp.float32)
    h = jnp.maximum(h + b1_ref[...], 0.0)
    o_ref[...] = (
        jnp.dot(
            h.astype(jnp.bfloat16),
            w2_ref[...],
            preferred_element_type=jnp.float32,
        )
        + b2_ref[...]
    )


def _tc_mlp(x, w1p, b1r, w2t, b2r):
    return pl.pallas_call(
        _mlp_body,
        out_shape=jax.ShapeDtypeStruct((B, NB), jnp.float32),
        grid=(B // BLK,),
        in_specs=[
            pl.BlockSpec((BLK, XW), lambda i: (i, 0)),
            pl.BlockSpec((XW, HID), lambda i: (0, 0)),
            pl.BlockSpec((1, HID), lambda i: (0, 0)),
            pl.BlockSpec((HID, NB), lambda i: (0, 0)),
            pl.BlockSpec((1, NB), lambda i: (0, 0)),
        ],
        out_specs=pl.BlockSpec((BLK, NB), lambda i: (i, 0)),
        compiler_params=pltpu.CompilerParams(
            dimension_semantics=("parallel",),
            vmem_limit_bytes=100 * 1024 * 1024,
        ),
    )(x, w1p, b1r, w2t, b2r)


def kernel(friends, enemies, map_idx, brawler_emb, map_emb, W1, b1, W2, b2):
    tab7 = jnp.concatenate(
        [
            jnp.pad(brawler_emb, ((0, 0), (16 * s, XW - EMB - 16 * s)))
            for s in range(6)
        ]
        + [jnp.pad(map_emb, ((0, 0), (16 * 6, XW - EMB - 16 * 6)))],
        axis=0,
    )

    off_f = (jnp.arange(3, dtype=jnp.int32) * NB)[:, None]
    off_e = ((3 + jnp.arange(3, dtype=jnp.int32)) * NB)[:, None]
    idx_sm = jnp.concatenate(
        [
            friends.astype(jnp.int32).T + off_f,
            enemies.astype(jnp.int32).T + off_e,
            map_idx.astype(jnp.int32).T + 6 * NB,
            jnp.zeros((1, B), jnp.int32),
        ],
        axis=0,
    )

    x = _sc_gather(tab7, idx_sm)

    w1p = jnp.concatenate(
        [W1.T, jnp.zeros((XW - W1.shape[1], HID), jnp.float32)], axis=0
    )
    b1r = b1.reshape(1, HID)
    w2t = W2.T.astype(jnp.bfloat16)
    b2r = b2.reshape(1, NB)
    return _tc_mlp(x, w1p, b1r, w2t, b2r)
